# baseline (speedup 1.0000x reference)
import functools

import jax
import jax.numpy as jnp
from jax import lax
from jax.experimental import pallas as pl
from jax.experimental.pallas import tpu as pltpu
from jax.experimental.pallas import tpu_sc as plsc

_NC = 2
_NS = 16
_NW = _NC * _NS

_CK = 128


def _sc_gather(idx_flat, tok_table, total_rows, embd):
    rows_per_w = total_rows // _NW
    n_chunk = rows_per_w // _CK
    mesh = plsc.VectorSubcoreMesh(core_axis_name="c", subcore_axis_name="s")

    @functools.partial(
        pl.kernel,
        mesh=mesh,
        out_type=jax.ShapeDtypeStruct((total_rows, embd), jnp.float32),
        scratch_types=[
            pltpu.VMEM((_CK,), jnp.int32),
            pltpu.VMEM((_CK, embd), jnp.float32),
            pltpu.SemaphoreType.DMA,
        ],
        compiler_params=pltpu.CompilerParams(use_tc_tiling_on_sc=False),
    )
    def gather_kernel(idx_hbm, tok_hbm, out_hbm, idx_v, rows_v, sem):
        wid = lax.axis_index("s") * _NC + lax.axis_index("c")
        base0 = wid * rows_per_w

        def chunk(c, carry):
            base = base0 + c * _CK
            pltpu.sync_copy(idx_hbm.at[pl.ds(base, _CK)], idx_v)
            pltpu.async_copy(tok_hbm.at[idx_v], rows_v, sem).wait()
            pltpu.sync_copy(rows_v, out_hbm.at[pl.ds(base, _CK)])
            return carry

        lax.fori_loop(0, n_chunk, chunk, 0)

    return gather_kernel(idx_flat, tok_table)


def _tc_head(emb, pos_table, W, b2, bm):
    total_rows, embd = emb.shape
    seq = pos_table.shape[0]
    vocab = W.shape[1]
    reps = bm // seq

    def body(emb_ref, pos_ref, w_ref, b_ref, out_ref):
        x = emb_ref[...] + jnp.tile(pos_ref[...], (reps, 1))
        out_ref[...] = (
            jnp.dot(x, w_ref[...], preferred_element_type=jnp.float32) + b_ref[...]
        )

    return pl.pallas_call(
        body,
        grid=(total_rows // bm,),
        in_specs=[
            pl.BlockSpec((bm, embd), lambda i: (i, 0)),
            pl.BlockSpec((seq, embd), lambda i: (0, 0)),
            pl.BlockSpec((embd, vocab), lambda i: (0, 0)),
            pl.BlockSpec((1, vocab), lambda i: (0, 0)),
        ],
        out_specs=pl.BlockSpec((bm, vocab), lambda i: (i, 0)),
        out_shape=jax.ShapeDtypeStruct((total_rows, vocab), jnp.float32),
        compiler_params=pltpu.CompilerParams(
            dimension_semantics=("parallel",),
        ),
    )(emb, pos_table, W, b2)


def kernel(idx, tok_table, pos_table, W, b):
    B, T = idx.shape
    vocab = W.shape[1]
    idx_flat = idx.reshape(-1).astype(jnp.int32)
    emb = _sc_gather(idx_flat, tok_table, B * T, tok_table.shape[1])
    logits = _tc_head(emb, pos_table, W, b.reshape(1, vocab), bm=512)
    return logits.reshape(B, T, vocab), None
